# Initial kernel scaffold; baseline (speedup 1.0000x reference)
import jax
import jax.numpy as jnp
from jax.experimental import pallas as pl


def kernel(v_event, v_item, v_category, W_event, W_item, W_category):
    raise NotImplementedError("write your pallas kernel here")


# speedup vs baseline: 3.8793x; 3.8793x over previous
import functools

import jax
import jax.numpy as jnp
from jax import lax
from jax.experimental import pallas as pl
from jax.experimental.pallas import tpu as pltpu
from jax.experimental.pallas import tpu_sc as plsc

_B, _L, _D, _V = 4096, 50, 128, 100000
_NF = 3
_NC, _NS = 2, 16
_NW = _NC * _NS
_BPW = _B // _NW
_NB = 4
_STEPS = _BPW // _NB
_ROWS_PER_B = _NF * _L


def _make_kernel():
    mesh = plsc.VectorSubcoreMesh(
        core_axis_name="c", subcore_axis_name="s",
        num_cores=_NC, num_subcores=_NS,
    )

    @functools.partial(
        pl.kernel,
        out_type=jax.ShapeDtypeStruct((_B * _ROWS_PER_B, _D), jnp.float32),
        mesh=mesh,
        scratch_types=[
            pltpu.VMEM((_NF, _BPW, _L), jnp.int32),
            pltpu.VMEM((_NB * _ROWS_PER_B, _D), jnp.float32),
            pltpu.SemaphoreType.DMA,
        ],
    )
    def emb(v_e, v_i, v_c, w_e, w_i, w_c, out, idx_v, rows_v, sem):
        wid = lax.axis_index("s") * _NC + lax.axis_index("c")
        b_base = wid * _BPW

        pltpu.sync_copy(v_e.at[pl.ds(b_base, _BPW)], idx_v.at[0])
        pltpu.sync_copy(v_i.at[pl.ds(b_base, _BPW)], idx_v.at[1])
        pltpu.sync_copy(v_c.at[pl.ds(b_base, _BPW)], idx_v.at[2])

        tables = (w_e, w_i, w_c)

        def step(s, _):
            copies = []
            for bl in range(_NB):
                for f in range(_NF):
                    dst = rows_v.at[pl.ds((bl * _NF + f) * _L, _L)]
                    copies.append(
                        pltpu.async_copy(
                            tables[f].at[idx_v.at[f, s * _NB + bl]], dst, sem
                        )
                    )
            for cp in copies:
                cp.wait()
            row0 = (b_base + s * _NB) * _ROWS_PER_B
            pltpu.sync_copy(rows_v, out.at[pl.ds(row0, _NB * _ROWS_PER_B)])
            return ()

        lax.fori_loop(0, _STEPS, step, ())

    return emb


_emb = _make_kernel()


def kernel(v_event, v_item, v_category, W_event, W_item, W_category):
    flat = _emb(v_event, v_item, v_category, W_event, W_item, W_category)
    return flat.reshape(_B, _ROWS_PER_B, _D)


# speedup vs baseline: 6.9980x; 1.8039x over previous
import functools

import jax
import jax.numpy as jnp
from jax import lax
from jax.experimental import pallas as pl
from jax.experimental.pallas import tpu as pltpu
from jax.experimental.pallas import tpu_sc as plsc

_B, _L, _D, _V = 4096, 50, 128, 100000
_NF = 3
_NC, _NS = 2, 16
_NW = _NC * _NS
_BPW = _B // _NW
_NB = 2
_STEPS = _BPW // _NB
_RPB = _NF * _L


def _make_kernel():
    mesh = plsc.VectorSubcoreMesh(
        core_axis_name="c", subcore_axis_name="s",
        num_cores=_NC, num_subcores=_NS,
    )

    @functools.partial(
        pl.kernel,
        out_type=jax.ShapeDtypeStruct((_B, _RPB, _D), jnp.float32),
        mesh=mesh,
        scratch_types=[
            pltpu.VMEM((_NF, _BPW, _L), jnp.int32),
            pltpu.VMEM((2, _NB, _RPB, _D), jnp.float32),
            pltpu.SemaphoreType.DMA,
            pltpu.SemaphoreType.DMA,
        ],
    )
    def emb(v_e, v_i, v_c, w_e, w_i, w_c, out, idx_v, rows_v, sem0, sem1):
        wid = lax.axis_index("s") * _NC + lax.axis_index("c")
        b_base = wid * _BPW
        sems = (sem0, sem1)
        tables = (w_e, w_i, w_c)

        pltpu.sync_copy(v_e.at[pl.ds(b_base, _BPW)], idx_v.at[0])
        pltpu.sync_copy(v_i.at[pl.ds(b_base, _BPW)], idx_v.at[1])
        pltpu.sync_copy(v_c.at[pl.ds(b_base, _BPW)], idx_v.at[2])

        def fire(s, k):
            for bl in range(_NB):
                for f in range(_NF):
                    dst = rows_v.at[k, bl, pl.ds(f * _L, _L)]
                    pltpu.async_copy(
                        tables[f].at[idx_v.at[f, s * _NB + bl]], dst, sems[k]
                    )

        def drain(k):
            pltpu.make_async_copy(
                out.at[pl.ds(0, _NB)], rows_v.at[k], sems[k]
            ).wait()

        def write(s, k):
            row_b = b_base + s * _NB
            pltpu.sync_copy(rows_v.at[k], out.at[pl.ds(row_b, _NB)])

        fire(0, 0)

        def body(i, _):
            for k in range(2):
                s = 2 * i + k
                drain(k)
                if k == 0:
                    fire(s + 1, 1)
                else:
                    @pl.when(i != _STEPS // 2 - 1)
                    def _():
                        fire(s + 1, 0)
                write(s, k)
            return ()

        lax.fori_loop(0, _STEPS // 2, body, ())

    return emb


_emb = _make_kernel()


def kernel(v_event, v_item, v_category, W_event, W_item, W_category):
    return _emb(v_event, v_item, v_category, W_event, W_item, W_category)
